# baseline (speedup 1.0000x reference)
import functools

import numpy as np
import jax
import jax.numpy as jnp
from jax.experimental import pallas as pl
from jax.experimental.pallas import tpu as pltpu
from jax.experimental.pallas import tpu_sc as plsc

N_BINS = 15
BATCH = 16384
NUM_CLASSES = 1000
BN = 1024
GRID = BATCH // BN
CHUNK = 256
NCH = BN // CHUNK

_BOUNDARIES = [
    0.0, 0.06666667014360428, 0.13333334028720856, 0.20000001788139343,
    0.2666666805744171, 0.3333333432674408, 0.40000003576278687,
    0.46666669845581055, 0.5333333611488342, 0.6000000238418579,
    0.6666666865348816, 0.7333333492279053, 0.8000000715255737,
    0.8666667342185974, 0.9333333969116211, 1.0,
]
_LOWER16 = np.array(_BOUNDARIES[:15] + [2.0], dtype=np.float32).reshape(1, 16)
_UPPER16 = np.array(_BOUNDARIES[1:16] + [3.0], dtype=np.float32).reshape(1, 16)


_NC = 2
_NS = 16
_NW = _NC * _NS
_SPW = BATCH // _NW
_GCH = 128
_NG = _SPW // _GCH


def _gather_body(logits_flat, labels_hbm, out_hbm, lab_v, idx_v, row_v, sem):
    wid = jax.lax.axis_index("s") * _NC + jax.lax.axis_index("c")
    base = wid * _SPW
    pltpu.sync_copy(labels_hbm.at[pl.ds(base, _SPW)], lab_v)
    for t in range(_NG):
        for u in range(_GCH // 16):
            o = t * _GCH + u * 16
            lab16 = lab_v[pl.ds(o, 16)]
            n16 = (base + o) + jax.lax.iota(jnp.int32, 16)
            idx_v[t, pl.ds(u * 16, 16)] = n16 * NUM_CLASSES + lab16
    for t in range(_NG):
        pltpu.async_copy(logits_flat.at[idx_v.at[t]], row_v.at[t], sem).wait()
        pltpu.sync_copy(row_v.at[t], out_hbm.at[pl.ds(base + t * _GCH, _GCH)])


def _gather_true(logits_flat, labels):
    k = functools.partial(
        pl.kernel,
        out_type=jax.ShapeDtypeStruct((BATCH,), jnp.float32),
        mesh=plsc.VectorSubcoreMesh(core_axis_name="c", subcore_axis_name="s"),
        scratch_types=[
            pltpu.VMEM((_SPW,), jnp.int32),
            pltpu.VMEM((_NG, _GCH), jnp.int32),
            pltpu.VMEM((_NG, _GCH), jnp.float32),
            pltpu.SemaphoreType.DMA,
        ],
    )(_gather_body)
    return k(logits_flat, labels)


def _ece_kernel(labels_ref, xt_ref, lower_ref, upper_ref, logits_ref,
                over_ref, under_ref, cnt_ref, csum_ref, acc_ref):
    i = pl.program_id(0)
    boundaries = _BOUNDARIES

    @pl.when(i == 0)
    def _():
        cnt_ref[...] = jnp.zeros((16, NUM_CLASSES), jnp.float32)
        csum_ref[...] = jnp.zeros((16, NUM_CLASSES), jnp.float32)
        acc_ref[...] = jnp.zeros((16, NUM_CLASSES), jnp.float32)

    lab = labels_ref[i]
    lab_col = lab.reshape(BN, 1)
    xt_col = xt_ref[i].reshape(BN, 1)
    cids = jax.lax.broadcasted_iota(jnp.int32, (1, NUM_CLASSES), 1)

    cnt0_acc = jnp.zeros((1, NUM_CLASSES), jnp.float32)
    csum0_acc = jnp.zeros((1, NUM_CLASSES), jnp.float32)
    onehot_chunks = []
    labmask_chunks = []

    for c in range(NCH):
        x = logits_ref[c * CHUNK:(c + 1) * CHUNK, :]
        m = jnp.max(x, axis=1, keepdims=True)
        e = jnp.exp(x - m)
        s = jnp.sum(e, axis=1, keepdims=True)
        conf = e / s

        cnt0_acc = cnt0_acc + jnp.sum(
            (conf > 0.0).astype(jnp.float32), axis=0, keepdims=True)
        csum0_acc = csum0_acc + jnp.sum(conf, axis=0, keepdims=True)

        cmax = 1.0 / jnp.min(s)
        for k in range(1, N_BINS):
            @pl.when(cmax > boundaries[k])
            def _(k=k, conf=conf):
                gt = (conf > boundaries[k]).astype(jnp.float32)
                cnt_ref[k:k + 1, :] += jnp.sum(gt, axis=0, keepdims=True)
                csum_ref[k:k + 1, :] += jnp.sum(
                    conf * gt, axis=0, keepdims=True)

        lab_mask = (lab_col[c * CHUNK:(c + 1) * CHUNK] == cids
                    ).astype(jnp.float32)
        conf_true = jnp.exp(xt_col[c * CHUNK:(c + 1) * CHUNK] - m) / s
        onehot_chunks.append(
            ((conf_true > lower_ref[...]) &
             (conf_true <= upper_ref[...])).astype(jnp.float32))
        labmask_chunks.append(lab_mask)

    cnt_ref[0:1, :] += cnt0_acc
    csum_ref[0:1, :] += csum0_acc

    onehot_bin = jnp.concatenate(onehot_chunks, axis=0)
    lab_mask_full = jnp.concatenate(labmask_chunks, axis=0)
    acc_ref[...] += jax.lax.dot_general(
        onehot_bin, lab_mask_full, (((0,), (0,)), ((), ())),
        preferred_element_type=jnp.float32)

    @pl.when(i == GRID - 1)
    def _():
        cum_cnt = cnt_ref[...]
        cum_csum = csum_ref[...]
        acc = acc_ref[...]
        zr = jnp.zeros((1, NUM_CLASSES), jnp.float32)
        count = cum_cnt - jnp.concatenate([cum_cnt[1:], zr], axis=0)
        conf_sum = cum_csum - jnp.concatenate([cum_csum[1:], zr], axis=0)
        denom = jnp.maximum(count, 1.0)
        diff = conf_sum / denom - acc / denom
        contrib = jnp.abs(diff) * (count * (1.0 / BATCH))
        num_classes_t = jnp.max(labels_ref[...]) + 1
        active = (cids < num_classes_t).astype(jnp.float32)
        nonempty = count > 0
        over_bc = jnp.where(nonempty & (diff > 0), contrib, 0.0) * active
        under_bc = jnp.where(nonempty & (diff <= 0), contrib, 0.0) * active
        over_ref[...] = jnp.broadcast_to(
            jnp.sum(over_bc, axis=1, keepdims=True), (16, 128))
        under_ref[...] = jnp.broadcast_to(
            jnp.sum(under_bc, axis=1, keepdims=True), (16, 128))


def kernel(logits, labels):
    labels2d = labels.reshape(GRID, BN)
    x_true = _gather_true(logits.reshape(-1), labels)
    over, under = pl.pallas_call(
        _ece_kernel,
        grid=(GRID,),
        in_specs=[
            pl.BlockSpec((GRID, BN), lambda i: (0, 0)),
            pl.BlockSpec((GRID, BN), lambda i: (0, 0)),
            pl.BlockSpec((1, 16), lambda i: (0, 0)),
            pl.BlockSpec((1, 16), lambda i: (0, 0)),
            pl.BlockSpec((BN, NUM_CLASSES), lambda i: (i, 0)),
        ],
        out_specs=[
            pl.BlockSpec((16, 128), lambda i: (0, 0)),
            pl.BlockSpec((16, 128), lambda i: (0, 0)),
        ],
        out_shape=[
            jax.ShapeDtypeStruct((16, 128), jnp.float32),
            jax.ShapeDtypeStruct((16, 128), jnp.float32),
        ],
        scratch_shapes=[
            pltpu.VMEM((16, NUM_CLASSES), jnp.float32),
            pltpu.VMEM((16, NUM_CLASSES), jnp.float32),
            pltpu.VMEM((16, NUM_CLASSES), jnp.float32),
        ],
    )(labels2d, x_true.reshape(GRID, BN),
      jnp.asarray(_LOWER16), jnp.asarray(_UPPER16), logits)
    boundaries = jnp.linspace(0.0, 1.0, N_BINS + 1)
    return over[:N_BINS, 0], under[:N_BINS, 0], boundaries[:-1]


# speedup vs baseline: 1.0774x; 1.0774x over previous
import numpy as np
import jax
import jax.numpy as jnp
from jax.experimental import pallas as pl
from jax.experimental.pallas import tpu as pltpu

N_BINS = 15
BATCH = 16384
NUM_CLASSES = 1000
BN = 1024
GRID = BATCH // BN
CHUNK = 256
NCH = BN // CHUNK

_BOUNDARIES = [
    0.0, 0.06666667014360428, 0.13333334028720856, 0.20000001788139343,
    0.2666666805744171, 0.3333333432674408, 0.40000003576278687,
    0.46666669845581055, 0.5333333611488342, 0.6000000238418579,
    0.6666666865348816, 0.7333333492279053, 0.8000000715255737,
    0.8666667342185974, 0.9333333969116211, 1.0,
]
_LOWER16 = np.array(_BOUNDARIES[:15] + [2.0], dtype=np.float32).reshape(1, 16)
_UPPER16 = np.array(_BOUNDARIES[1:16] + [3.0], dtype=np.float32).reshape(1, 16)


def _ece_kernel(labels_ref, lower_ref, upper_ref, logits_ref,
                over_ref, under_ref, cnt_ref, csum_ref, acc_ref):
    i = pl.program_id(0)
    boundaries = _BOUNDARIES

    @pl.when(i == 0)
    def _():
        cnt_ref[...] = jnp.zeros((16, NUM_CLASSES), jnp.float32)
        csum_ref[...] = jnp.zeros((16, NUM_CLASSES), jnp.float32)
        acc_ref[...] = jnp.zeros((16, NUM_CLASSES), jnp.float32)

    lab = labels_ref[i]
    lab_col = lab.reshape(BN, 1)
    cids = jax.lax.broadcasted_iota(jnp.int32, (1, NUM_CLASSES), 1)

    cnt0_acc = jnp.zeros((1, NUM_CLASSES), jnp.float32)
    csum0_acc = jnp.zeros((1, NUM_CLASSES), jnp.float32)
    onehot_chunks = []
    labmask_chunks = []

    for c in range(NCH):
        x = logits_ref[c * CHUNK:(c + 1) * CHUNK, :]
        m = jnp.max(x, axis=1, keepdims=True)
        e = jnp.exp(x - m)
        s = jnp.sum(e, axis=1, keepdims=True)
        conf = e / s

        cnt0_acc = cnt0_acc + jnp.sum(
            (conf > 0.0).astype(jnp.float32), axis=0, keepdims=True)
        csum0_acc = csum0_acc + jnp.sum(conf, axis=0, keepdims=True)

        cmax = 1.0 / jnp.min(s)
        for k in range(1, N_BINS):
            @pl.when(cmax > boundaries[k])
            def _(k=k, conf=conf):
                gt = (conf > boundaries[k]).astype(jnp.float32)
                cnt_ref[k:k + 1, :] += jnp.sum(gt, axis=0, keepdims=True)
                csum_ref[k:k + 1, :] += jnp.sum(
                    conf * gt, axis=0, keepdims=True)

        lab_mask = (lab_col[c * CHUNK:(c + 1) * CHUNK] == cids
                    ).astype(jnp.float32)
        conf_true = jnp.sum(conf * lab_mask, axis=1, keepdims=True)
        onehot_chunks.append(
            ((conf_true > lower_ref[...]) &
             (conf_true <= upper_ref[...])).astype(jnp.float32))
        labmask_chunks.append(lab_mask)

    cnt_ref[0:1, :] += cnt0_acc
    csum_ref[0:1, :] += csum0_acc

    onehot_bin = jnp.concatenate(onehot_chunks, axis=0)
    lab_mask_full = jnp.concatenate(labmask_chunks, axis=0)
    acc_ref[...] += jax.lax.dot_general(
        onehot_bin, lab_mask_full, (((0,), (0,)), ((), ())),
        preferred_element_type=jnp.float32)

    @pl.when(i == GRID - 1)
    def _():
        cum_cnt = cnt_ref[...]
        cum_csum = csum_ref[...]
        acc = acc_ref[...]
        zr = jnp.zeros((1, NUM_CLASSES), jnp.float32)
        count = cum_cnt - jnp.concatenate([cum_cnt[1:], zr], axis=0)
        conf_sum = cum_csum - jnp.concatenate([cum_csum[1:], zr], axis=0)
        denom = jnp.maximum(count, 1.0)
        diff = conf_sum / denom - acc / denom
        contrib = jnp.abs(diff) * (count * (1.0 / BATCH))
        num_classes_t = jnp.max(labels_ref[...]) + 1
        active = (cids < num_classes_t).astype(jnp.float32)
        nonempty = count > 0
        over_bc = jnp.where(nonempty & (diff > 0), contrib, 0.0) * active
        under_bc = jnp.where(nonempty & (diff <= 0), contrib, 0.0) * active
        over_ref[...] = jnp.broadcast_to(
            jnp.sum(over_bc, axis=1, keepdims=True), (16, 128))
        under_ref[...] = jnp.broadcast_to(
            jnp.sum(under_bc, axis=1, keepdims=True), (16, 128))


def kernel(logits, labels):
    labels2d = labels.reshape(GRID, BN)
    over, under = pl.pallas_call(
        _ece_kernel,
        grid=(GRID,),
        in_specs=[
            pl.BlockSpec((GRID, BN), lambda i: (0, 0)),
            pl.BlockSpec((1, 16), lambda i: (0, 0)),
            pl.BlockSpec((1, 16), lambda i: (0, 0)),
            pl.BlockSpec((BN, NUM_CLASSES), lambda i: (i, 0)),
        ],
        out_specs=[
            pl.BlockSpec((16, 128), lambda i: (0, 0)),
            pl.BlockSpec((16, 128), lambda i: (0, 0)),
        ],
        out_shape=[
            jax.ShapeDtypeStruct((16, 128), jnp.float32),
            jax.ShapeDtypeStruct((16, 128), jnp.float32),
        ],
        scratch_shapes=[
            pltpu.VMEM((16, NUM_CLASSES), jnp.float32),
            pltpu.VMEM((16, NUM_CLASSES), jnp.float32),
            pltpu.VMEM((16, NUM_CLASSES), jnp.float32),
        ],
    )(labels2d, jnp.asarray(_LOWER16), jnp.asarray(_UPPER16), logits)
    boundaries = jnp.linspace(0.0, 1.0, N_BINS + 1)
    return over[:N_BINS, 0], under[:N_BINS, 0], boundaries[:-1]
